# Initial kernel scaffold; baseline (speedup 1.0000x reference)
import jax
import jax.numpy as jnp
from jax.experimental import pallas as pl


def kernel(x, edge_index, W1, b1, W2, b2, Wfc, bfc):
    raise NotImplementedError("write your pallas kernel here")


# speedup vs baseline: 26.9975x; 26.9975x over previous
import functools

import jax
import jax.numpy as jnp
from jax import lax
from jax.experimental import pallas as pl
from jax.experimental.pallas import tpu as pltpu
from jax.experimental.pallas import tpu_sc as plsc

N = 10000
NP = 10240
E = 320000
D = 128
NC = 2
NS = 16
NW = NC * NS
EPW = E // NW
CH = 125
NCH = EPW // CH
RPT = NP // NS
BLK = 256
GRID = NP // BLK

_mesh = plsc.VectorSubcoreMesh(
    core_axis_name="c", subcore_axis_name="s", num_cores=NC, num_subcores=NS)
_sc_params = pltpu.CompilerParams(needs_layout_passes=False)


@functools.partial(
    pl.kernel,
    out_type=jax.ShapeDtypeStruct((NC, NP), jnp.float32),
    mesh=_mesh,
    scratch_types=[
        pltpu.VMEM((EPW // 16, 16), jnp.int32),
        pltpu.VMEM((NP,), jnp.float32),
        pltpu.VMEM((NS, RPT), jnp.float32),
        pltpu.VMEM_SHARED((NS, NP), jnp.float32),
    ],
    compiler_params=_sc_params,
)
def _deg_kernel(dst_hbm, out_hbm, didx, pdeg, tmp, slab):
  c = lax.axis_index("c")
  s = lax.axis_index("s")
  wid = s * NC + c

  pltpu.sync_copy(dst_hbm.at[wid], didx)

  zero16 = jnp.zeros((16,), jnp.float32)

  def _zero(i, _):
    pdeg[pl.ds(i * 16, 16)] = zero16
    return 0
  lax.fori_loop(0, NP // 16, _zero, 0)

  ones16 = jnp.ones((16,), jnp.float32)

  def _count(i, _):
    idx = didx[i]
    plsc.addupdate_scatter(pdeg, [idx], ones16)
    return 0
  lax.fori_loop(0, EPW // 16, _count, 0)

  pltpu.sync_copy(pdeg, slab.at[s])
  plsc.subcore_barrier()
  for r in range(NS):
    pltpu.sync_copy(slab.at[r, pl.ds(s * RPT, RPT)], tmp.at[r])

  def _reduce(k, _):
    v = tmp[0, pl.ds(k * 16, 16)]
    for r in range(1, NS):
      v = v + tmp[r, pl.ds(k * 16, 16)]
    pdeg[pl.ds(k * 16, 16)] = v
    return 0
  lax.fori_loop(0, RPT // 16, _reduce, 0)

  pltpu.sync_copy(pdeg.at[pl.ds(0, RPT)], out_hbm.at[c, pl.ds(s * RPT, RPT)])


@functools.partial(
    pl.kernel,
    out_type=jax.ShapeDtypeStruct((NC, NP, D), jnp.float32),
    mesh=_mesh,
    scratch_types=[
        pltpu.VMEM((NCH // 2, CH), jnp.int32),
        pltpu.VMEM((NCH // 2, CH), jnp.int32),
        pltpu.VMEM((CH, D), jnp.float32),
        pltpu.VMEM((CH, D), jnp.float32),
        pltpu.VMEM((32, D), jnp.float32),
        pltpu.VMEM_SHARED((NP, D), jnp.float32),
        pltpu.SemaphoreType.DMA,
        pltpu.SemaphoreType.DMA,
    ],
    compiler_params=_sc_params,
)
def _agg_kernel(u_hbm, src_hbm, dst_hbm, out_hbm,
                sidx, didx, bufa, bufb, zbuf, acc, sema, semb):
  c = lax.axis_index("c")
  s = lax.axis_index("s")
  wid = s * NC + c

  zero16 = jnp.zeros((16,), jnp.float32)

  def _zrow(i, _):
    for j in range(D // 16):
      zbuf[i, pl.ds(j * 16, 16)] = zero16
    return 0
  lax.fori_loop(0, 32, _zrow, 0)

  def _zacc(k, _):
    pltpu.sync_copy(zbuf, acc.at[pl.ds(s * RPT + k * 32, 32)])
    return 0
  lax.fori_loop(0, RPT // 32, _zacc, 0)

  plsc.subcore_barrier()

  half = NCH // 2
  for h in range(2):
    pltpu.sync_copy(src_hbm.at[wid, pl.ds(h * half, half)], sidx)
    pltpu.sync_copy(dst_hbm.at[wid, pl.ds(h * half, half)], didx)
    pltpu.make_async_copy(u_hbm.at[sidx.at[0]], bufa, sema).start()

    def _body(i, _):
      j0 = 2 * i
      j1 = 2 * i + 1
      pltpu.make_async_copy(u_hbm.at[sidx.at[j0]], bufa, sema).wait()
      pltpu.make_async_copy(u_hbm.at[sidx.at[j1]], bufb, semb).start()
      pltpu.sync_copy(bufa, acc.at[didx.at[j0]], add=True)
      pltpu.make_async_copy(u_hbm.at[sidx.at[j1]], bufb, semb).wait()

      @pl.when(i < half // 2 - 1)
      def _():
        pltpu.make_async_copy(u_hbm.at[sidx.at[j0 + 2]], bufa, sema).start()

      pltpu.sync_copy(bufb, acc.at[didx.at[j1]], add=True)
      return 0
    lax.fori_loop(0, half // 2, _body, 0)

  plsc.subcore_barrier()
  pltpu.sync_copy(acc.at[pl.ds(s * RPT, RPT)],
                  out_hbm.at[c, pl.ds(s * RPT, RPT)])


def _scale_body(pdt_ref, x_ref, o_ref):
  deg = 1.0 + pdt_ref[:, 0] + pdt_ref[:, 1]
  dinv = lax.rsqrt(deg)
  o_ref[...] = x_ref[...] * dinv[:, None]


def _mid_body(pdt_ref, s_ref, u_ref, w1_ref, b1_ref, w2_ref, o_ref):
  deg = 1.0 + pdt_ref[:, 0] + pdt_ref[:, 1]
  dinv = lax.rsqrt(deg)
  agg = (s_ref[0] + s_ref[1] + u_ref[...]) * dinv[:, None]
  h1 = jnp.maximum(
      jnp.dot(agg, w1_ref[...], preferred_element_type=jnp.float32)
      + b1_ref[...], 0.0)
  z2 = jnp.dot(h1, w2_ref[...], preferred_element_type=jnp.float32)
  o_ref[...] = z2 * dinv[:, None]


def _head_body(pdt_ref, s_ref, u_ref, b2_ref, wfc_ref, bfc_ref, o_ref):
  deg = 1.0 + pdt_ref[:, 0] + pdt_ref[:, 1]
  dinv = lax.rsqrt(deg)
  h2 = jnp.maximum(
      (s_ref[0] + s_ref[1] + u_ref[...]) * dinv[:, None] + b2_ref[...], 0.0)
  o_ref[...] = (jnp.dot(h2, wfc_ref[...], preferred_element_type=jnp.float32)
                + bfc_ref[...])


_pdt_spec = pl.BlockSpec((BLK, NC), lambda i: (i, 0))
_row_spec = pl.BlockSpec((BLK, D), lambda i: (i, 0))
_s_spec = pl.BlockSpec((NC, BLK, D), lambda i: (0, i, 0))


def _scale_call(pdt, xp):
  return pl.pallas_call(
      _scale_body,
      grid=(GRID,),
      in_specs=[_pdt_spec, _row_spec],
      out_specs=_row_spec,
      out_shape=jax.ShapeDtypeStruct((NP, D), jnp.float32),
  )(pdt, xp)


def _mid_call(pdt, s1, u1, W1, b1, W2):
  h1w = W1.shape[1]
  return pl.pallas_call(
      _mid_body,
      grid=(GRID,),
      in_specs=[
          _pdt_spec, _s_spec, _row_spec,
          pl.BlockSpec((D, h1w), lambda i: (0, 0)),
          pl.BlockSpec((1, h1w), lambda i: (0, 0)),
          pl.BlockSpec((h1w, D), lambda i: (0, 0)),
      ],
      out_specs=_row_spec,
      out_shape=jax.ShapeDtypeStruct((NP, D), jnp.float32),
  )(pdt, s1, u1, W1, b1.reshape(1, h1w), W2)


def _head_call(pdt, s2, u2, b2, Wfc, bfc):
  return pl.pallas_call(
      _head_body,
      grid=(GRID,),
      in_specs=[
          _pdt_spec, _s_spec, _row_spec,
          pl.BlockSpec((1, D), lambda i: (0, 0)),
          pl.BlockSpec((D, 1), lambda i: (0, 0)),
          pl.BlockSpec((1, 1), lambda i: (0, 0)),
      ],
      out_specs=pl.BlockSpec((BLK, 1), lambda i: (i, 0)),
      out_shape=jax.ShapeDtypeStruct((NP, 1), jnp.float32),
  )(pdt, s2, u2, b2.reshape(1, D), Wfc, bfc.reshape(1, 1))


@jax.jit
def kernel(x, edge_index, W1, b1, W2, b2, Wfc, bfc):
  src = edge_index[0].reshape(NW, NCH, CH)
  dst = edge_index[1].reshape(NW, NCH, CH)
  dst16 = edge_index[1].reshape(NW, EPW // 16, 16)
  xp = jnp.pad(x, ((0, NP - N), (0, 0)))

  pd = _deg_kernel(dst16)
  pdt = pd.T
  u1 = _scale_call(pdt, xp)
  s1 = _agg_kernel(u1, src, dst)
  u2 = _mid_call(pdt, s1, u1, W1, b1, W2)
  s2 = _agg_kernel(u2, src, dst)
  out = _head_call(pdt, s2, u2, b2, Wfc, bfc)
  return out[:N]
